# baseline (speedup 1.0000x reference)
import functools

import jax
import jax.numpy as jnp
from jax import lax
from jax.experimental import pallas as pl
from jax.experimental.pallas import tpu as pltpu
from jax.experimental.pallas import tpu_sc as plsc

N_NODES = 100000
N_PAD = 102400
N_EDGES = 3200000

NC = 2
NS = 16
NW = NC * NS
G = 128
GP = 8
NG = N_EDGES // G
GROUPS_PER_W = NG // NW
CHUNKS = GROUPS_PER_W // GP
TAIL = GROUPS_PER_W - CHUNKS * GP
NG_EVEN = GROUPS_PER_W * NW

STG = N_PAD // NS

_mesh = plsc.VectorSubcoreMesh(core_axis_name="c", subcore_axis_name="s")
_sc_params = pltpu.CompilerParams(use_tc_tiling_on_sc=False)


def _stage_slices(sid, copy_fn):
    copy_fn(sid * STG, STG)


@functools.partial(
    pl.kernel,
    out_type=jax.ShapeDtypeStruct((NC, N_PAD, 8), jnp.float32),
    mesh=_mesh,
    scratch_types=[
        pltpu.VMEM_SHARED((N_PAD, 8), jnp.float32),
        pltpu.VMEM_SHARED((N_PAD, 8), jnp.float32),
        pltpu.VMEM((GP, G), jnp.int32),
        pltpu.VMEM((GP, G), jnp.int32),
        pltpu.VMEM((GP, G, 8), jnp.float32),
        pltpu.SemaphoreType.DMA,
        pltpu.SemaphoreType.DMA,
    ],
    compiler_params=_sc_params,
)
def _sc_pass1(x4_hbm, edges_hbm, zeros_hbm, out_hbm,
              table_sh, accum_sh, src_v, dst_v, rows_v, gsem, ssem):
    c = lax.axis_index("c")
    s = lax.axis_index("s")
    w = s * NC + c

    def stage(off, sz):
        pltpu.sync_copy(x4_hbm.at[pl.ds(off, sz)], table_sh.at[pl.ds(off, sz)])
        pltpu.sync_copy(zeros_hbm.at[pl.ds(off, sz)], accum_sh.at[pl.ds(off, sz)])

    _stage_slices(s, stage)
    plsc.subcore_barrier()

    def run_groups(n):
        gathers = [
            pltpu.async_copy(table_sh.at[src_v.at[j]], rows_v.at[j], gsem)
            for j in range(n)
        ]
        for cp in gathers:
            cp.wait()
        scatters = [
            pltpu.async_copy(rows_v.at[j], accum_sh.at[dst_v.at[j]], ssem,
                             add=True)
            for j in range(n)
        ]
        for cp in scatters:
            cp.wait()

    base = w * GROUPS_PER_W

    def chunk(k, carry):
        goff = base + k * GP
        pltpu.sync_copy(edges_hbm.at[pl.ds(goff, GP), 0], src_v)
        pltpu.sync_copy(edges_hbm.at[pl.ds(goff, GP), 1], dst_v)
        run_groups(GP)
        return carry

    lax.fori_loop(0, CHUNKS, chunk, 0)

    goff_t = base + CHUNKS * GP
    pltpu.sync_copy(edges_hbm.at[pl.ds(goff_t, TAIL), 0],
                    src_v.at[pl.ds(0, TAIL)])
    pltpu.sync_copy(edges_hbm.at[pl.ds(goff_t, TAIL), 1],
                    dst_v.at[pl.ds(0, TAIL)])

    @pl.when(w < NG - NG_EVEN)
    def _():
        pltpu.sync_copy(edges_hbm.at[NG_EVEN + w, 0], src_v.at[TAIL])
        pltpu.sync_copy(edges_hbm.at[NG_EVEN + w, 1], dst_v.at[TAIL])

    run_groups(TAIL)

    @pl.when(w < NG - NG_EVEN)
    def _():
        pltpu.async_copy(table_sh.at[src_v.at[TAIL]], rows_v.at[TAIL],
                         gsem).wait()
        pltpu.async_copy(rows_v.at[TAIL], accum_sh.at[dst_v.at[TAIL]], ssem,
                         add=True).wait()
    plsc.subcore_barrier()

    def writeback(off, sz):
        pltpu.sync_copy(accum_sh.at[pl.ds(off, sz)],
                        out_hbm.at[c, pl.ds(off, sz)])

    _stage_slices(s, writeback)


@functools.partial(
    pl.kernel,
    out_type=jax.ShapeDtypeStruct((NC, N_PAD, 8), jnp.float32),
    mesh=_mesh,
    scratch_types=[
        pltpu.VMEM_SHARED((N_PAD, 8), jnp.float32),
        pltpu.VMEM_SHARED((N_PAD, 8), jnp.float32),
        pltpu.VMEM((GP, G), jnp.int32),
        pltpu.VMEM((GP, G), jnp.int32),
        pltpu.VMEM((GP, G, 8), jnp.float32),
        pltpu.SemaphoreType.DMA,
        pltpu.SemaphoreType.DMA,
    ],
    compiler_params=_sc_params,
)
def _sc_pass2(z_hbm, edges_hbm, zeros_hbm, out_hbm,
              table_sh, accum_sh, src_v, dst_v, rows_v, gsem, ssem):
    c = lax.axis_index("c")
    s = lax.axis_index("s")
    w = s * NC + c

    def stage(off, sz):
        pltpu.sync_copy(z_hbm.at[pl.ds(off, sz)], table_sh.at[pl.ds(off, sz)])
        pltpu.sync_copy(zeros_hbm.at[pl.ds(off, sz)], accum_sh.at[pl.ds(off, sz)])

    _stage_slices(s, stage)
    plsc.subcore_barrier()

    def run_groups(n):
        gathers = [
            pltpu.async_copy(table_sh.at[src_v.at[j]], rows_v.at[j], gsem)
            for j in range(n)
        ]
        for cp in gathers:
            cp.wait()
        scatters = [
            pltpu.async_copy(rows_v.at[j], accum_sh.at[dst_v.at[j]], ssem,
                             add=True)
            for j in range(n)
        ]
        for cp in scatters:
            cp.wait()

    base = w * GROUPS_PER_W

    def chunk(k, carry):
        goff = base + k * GP
        pltpu.sync_copy(edges_hbm.at[pl.ds(goff, GP), 0], src_v)
        pltpu.sync_copy(edges_hbm.at[pl.ds(goff, GP), 1], dst_v)
        run_groups(GP)
        return carry

    lax.fori_loop(0, CHUNKS, chunk, 0)

    goff_t = base + CHUNKS * GP
    pltpu.sync_copy(edges_hbm.at[pl.ds(goff_t, TAIL), 0],
                    src_v.at[pl.ds(0, TAIL)])
    pltpu.sync_copy(edges_hbm.at[pl.ds(goff_t, TAIL), 1],
                    dst_v.at[pl.ds(0, TAIL)])

    @pl.when(w < NG - NG_EVEN)
    def _():
        pltpu.sync_copy(edges_hbm.at[NG_EVEN + w, 0], src_v.at[TAIL])
        pltpu.sync_copy(edges_hbm.at[NG_EVEN + w, 1], dst_v.at[TAIL])

    run_groups(TAIL)

    @pl.when(w < NG - NG_EVEN)
    def _():
        pltpu.async_copy(table_sh.at[src_v.at[TAIL]], rows_v.at[TAIL],
                         gsem).wait()
        pltpu.async_copy(rows_v.at[TAIL], accum_sh.at[dst_v.at[TAIL]], ssem,
                         add=True).wait()
    plsc.subcore_barrier()

    def writeback(off, sz):
        pltpu.sync_copy(accum_sh.at[pl.ds(off, sz)],
                        out_hbm.at[c, pl.ds(off, sz)])

    _stage_slices(s, writeback)


NR = 800
BRR = 16
F = 16


def _tc_dense1_body(p_ref, x_ref, wl1_ref, bl1_ref, wr1_ref, wl2_ref,
                    bl2_ref, wr2_ref, z8_ref, r_ref, deg_ref):
    p = p_ref[...]
    ssum = p[0] + p[1]
    deg = jnp.maximum(ssum[:, :, 2], 1.0)
    m0 = ssum[:, :, 0] / deg
    m1 = ssum[:, :, 1] / deg
    xv = x_ref[...]
    x0 = xv[:, :, 0]
    x1 = xv[:, :, 1]
    zacc = jnp.zeros_like(m0)
    racc = jnp.zeros_like(m0)
    for f in range(F):
        hf = (m0 * wl1_ref[f, 0] + m1 * wl1_ref[f, 1] + bl1_ref[f]
              + x0 * wr1_ref[f, 0] + x1 * wr1_ref[f, 1])
        hf = jnp.maximum(hf, 0.0)
        zacc = zacc + hf * wl2_ref[0, f]
        racc = racc + hf * wr2_ref[0, f]
    z8_ref[...] = jnp.broadcast_to(zacc[:, :, None], (BRR, 128, 8))
    r_ref[...] = racc + bl2_ref[0]
    deg_ref[...] = deg


def _tc_dense2_body(pz_ref, deg_ref, r_ref, out_ref):
    pz = pz_ref[...]
    aggz = pz[0, :, :, 0] + pz[1, :, :, 0]
    out_ref[...] = jax.nn.sigmoid(aggz / deg_ref[...] + r_ref[...])


def kernel(x, edge_index, Wl1, bl1, Wr1, Wl2, bl2, Wr2):
    f32 = jnp.float32
    x = x.astype(f32)
    e3 = (edge_index.astype(jnp.int32)
          .reshape(2, NG, G).transpose(1, 0, 2))
    x8 = jnp.concatenate(
        [x, jnp.ones((N_NODES, 1), f32), jnp.zeros((N_NODES, 5), f32)], axis=1)
    x8 = jnp.pad(x8, ((0, N_PAD - N_NODES), (0, 0)))
    zeros8 = jnp.zeros((N_PAD, 8), f32)

    part1 = _sc_pass1(x8, e3, zeros8)

    smem = pltpu.SMEM
    grid = (NR // BRR,)
    z8, r, deg = pl.pallas_call(
        _tc_dense1_body,
        grid=grid,
        in_specs=[
            pl.BlockSpec((NC, BRR, 128, 8), lambda i: (0, i, 0, 0)),
            pl.BlockSpec((BRR, 128, 8), lambda i: (i, 0, 0)),
            pl.BlockSpec(memory_space=smem),
            pl.BlockSpec(memory_space=smem),
            pl.BlockSpec(memory_space=smem),
            pl.BlockSpec(memory_space=smem),
            pl.BlockSpec(memory_space=smem),
            pl.BlockSpec(memory_space=smem),
        ],
        out_specs=[
            pl.BlockSpec((BRR, 128, 8), lambda i: (i, 0, 0)),
            pl.BlockSpec((BRR, 128), lambda i: (i, 0)),
            pl.BlockSpec((BRR, 128), lambda i: (i, 0)),
        ],
        out_shape=[
            jax.ShapeDtypeStruct((NR, 128, 8), f32),
            jax.ShapeDtypeStruct((NR, 128), f32),
            jax.ShapeDtypeStruct((NR, 128), f32),
        ],
    )(part1.reshape(NC, NR, 128, 8), x8.reshape(NR, 128, 8),
      Wl1, bl1, Wr1, Wl2, bl2, Wr2)

    part2 = _sc_pass2(z8.reshape(N_PAD, 8), e3, zeros8)

    out = pl.pallas_call(
        _tc_dense2_body,
        grid=grid,
        in_specs=[
            pl.BlockSpec((NC, BRR, 128, 8), lambda i: (0, i, 0, 0)),
            pl.BlockSpec((BRR, 128), lambda i: (i, 0)),
            pl.BlockSpec((BRR, 128), lambda i: (i, 0)),
        ],
        out_specs=pl.BlockSpec((BRR, 128), lambda i: (i, 0)),
        out_shape=jax.ShapeDtypeStruct((NR, 128), f32),
    )(part2.reshape(NC, NR, 128, 8), deg, r)
    return out.reshape(N_PAD)[:N_NODES]


# speedup vs baseline: 1.6089x; 1.6089x over previous
import functools

import jax
import jax.numpy as jnp
from jax import lax
from jax.experimental import pallas as pl
from jax.experimental.pallas import tpu as pltpu
from jax.experimental.pallas import tpu_sc as plsc

N_NODES = 100000
N_PAD = 102400
N_EDGES = 3200000

NC = 2
NS = 16
NW = NC * NS
G = 128
GP = 8
NG = N_EDGES // G
GROUPS_PER_W = NG // NW
CHUNKS = GROUPS_PER_W // GP
TAIL = GROUPS_PER_W - CHUNKS * GP
NG_EVEN = GROUPS_PER_W * NW

STG = N_PAD // NS

_mesh = plsc.VectorSubcoreMesh(core_axis_name="c", subcore_axis_name="s")
_sc_params = pltpu.CompilerParams(use_tc_tiling_on_sc=False)


def _stage_slices(sid, copy_fn):
    copy_fn(sid * STG, STG)


@functools.partial(
    pl.kernel,
    out_type=jax.ShapeDtypeStruct((NC, N_PAD, 8), jnp.float32),
    mesh=_mesh,
    scratch_types=[
        pltpu.VMEM_SHARED((N_PAD, 8), jnp.float32),
        pltpu.VMEM_SHARED((N_PAD, 8), jnp.float32),
        pltpu.VMEM((GP, G), jnp.int32),
        pltpu.VMEM((GP, G), jnp.int32),
        pltpu.VMEM((GP, G, 8), jnp.float32),
        pltpu.SemaphoreType.DMA,
        pltpu.SemaphoreType.DMA,
    ],
    compiler_params=_sc_params,
)
def _sc_pass1(x4_hbm, edges_hbm, zeros_hbm, out_hbm,
              table_sh, accum_sh, src_v, dst_v, rows_v, gsem, ssem):
    c = lax.axis_index("c")
    s = lax.axis_index("s")
    w = s * NC + c

    def stage(off, sz):
        pltpu.sync_copy(x4_hbm.at[pl.ds(off, sz)], table_sh.at[pl.ds(off, sz)])
        pltpu.sync_copy(zeros_hbm.at[pl.ds(off, sz)], accum_sh.at[pl.ds(off, sz)])

    _stage_slices(s, stage)
    plsc.subcore_barrier()

    def run_groups(n):
        gathers = [
            pltpu.async_copy(table_sh.at[src_v.at[j]], rows_v.at[j], gsem)
            for j in range(n)
        ]
        for cp in gathers:
            cp.wait()
        scatters = [
            pltpu.async_copy(rows_v.at[j], accum_sh.at[dst_v.at[j]], ssem,
                             add=True)
            for j in range(n)
        ]
        for cp in scatters:
            cp.wait()

    base = w * GROUPS_PER_W

    def chunk(k, carry):
        goff = base + k * GP
        pltpu.sync_copy(edges_hbm.at[pl.ds(goff, GP), 0], src_v)
        pltpu.sync_copy(edges_hbm.at[pl.ds(goff, GP), 1], dst_v)
        run_groups(GP)
        return carry

    lax.fori_loop(0, CHUNKS, chunk, 0)

    goff_t = base + CHUNKS * GP
    pltpu.sync_copy(edges_hbm.at[pl.ds(goff_t, TAIL), 0],
                    src_v.at[pl.ds(0, TAIL)])
    pltpu.sync_copy(edges_hbm.at[pl.ds(goff_t, TAIL), 1],
                    dst_v.at[pl.ds(0, TAIL)])

    @pl.when(w < NG - NG_EVEN)
    def _():
        pltpu.sync_copy(edges_hbm.at[NG_EVEN + w, 0], src_v.at[TAIL])
        pltpu.sync_copy(edges_hbm.at[NG_EVEN + w, 1], dst_v.at[TAIL])

    run_groups(TAIL)

    @pl.when(w < NG - NG_EVEN)
    def _():
        pltpu.async_copy(table_sh.at[src_v.at[TAIL]], rows_v.at[TAIL],
                         gsem).wait()
        pltpu.async_copy(rows_v.at[TAIL], accum_sh.at[dst_v.at[TAIL]], ssem,
                         add=True).wait()
    plsc.subcore_barrier()

    def writeback(off, sz):
        pltpu.sync_copy(accum_sh.at[pl.ds(off, sz)],
                        out_hbm.at[c, pl.ds(off, sz)])

    _stage_slices(s, writeback)


@functools.partial(
    pl.kernel,
    out_type=jax.ShapeDtypeStruct((NC, N_PAD, 8), jnp.float32),
    mesh=_mesh,
    scratch_types=[
        pltpu.VMEM_SHARED((N_PAD, 8), jnp.float32),
        pltpu.VMEM_SHARED((N_PAD, 8), jnp.float32),
        pltpu.VMEM((GP, G), jnp.int32),
        pltpu.VMEM((GP, G), jnp.int32),
        pltpu.VMEM((GP, G, 8), jnp.float32),
        pltpu.SemaphoreType.DMA,
        pltpu.SemaphoreType.DMA,
    ],
    compiler_params=_sc_params,
)
def _sc_pass2(z_hbm, edges_hbm, zeros_hbm, out_hbm,
              table_sh, accum_sh, src_v, dst_v, rows_v, gsem, ssem):
    c = lax.axis_index("c")
    s = lax.axis_index("s")
    w = s * NC + c

    def stage(off, sz):
        pltpu.sync_copy(z_hbm.at[pl.ds(off, sz)], table_sh.at[pl.ds(off, sz)])
        pltpu.sync_copy(zeros_hbm.at[pl.ds(off, sz)], accum_sh.at[pl.ds(off, sz)])

    _stage_slices(s, stage)
    plsc.subcore_barrier()

    def run_groups(n):
        gathers = [
            pltpu.async_copy(table_sh.at[src_v.at[j]], rows_v.at[j], gsem)
            for j in range(n)
        ]
        for cp in gathers:
            cp.wait()
        scatters = [
            pltpu.async_copy(rows_v.at[j], accum_sh.at[dst_v.at[j]], ssem,
                             add=True)
            for j in range(n)
        ]
        for cp in scatters:
            cp.wait()

    base = w * GROUPS_PER_W

    def chunk(k, carry):
        goff = base + k * GP
        pltpu.sync_copy(edges_hbm.at[pl.ds(goff, GP), 0], src_v)
        pltpu.sync_copy(edges_hbm.at[pl.ds(goff, GP), 1], dst_v)
        run_groups(GP)
        return carry

    lax.fori_loop(0, CHUNKS, chunk, 0)

    goff_t = base + CHUNKS * GP
    pltpu.sync_copy(edges_hbm.at[pl.ds(goff_t, TAIL), 0],
                    src_v.at[pl.ds(0, TAIL)])
    pltpu.sync_copy(edges_hbm.at[pl.ds(goff_t, TAIL), 1],
                    dst_v.at[pl.ds(0, TAIL)])

    @pl.when(w < NG - NG_EVEN)
    def _():
        pltpu.sync_copy(edges_hbm.at[NG_EVEN + w, 0], src_v.at[TAIL])
        pltpu.sync_copy(edges_hbm.at[NG_EVEN + w, 1], dst_v.at[TAIL])

    run_groups(TAIL)

    @pl.when(w < NG - NG_EVEN)
    def _():
        pltpu.async_copy(table_sh.at[src_v.at[TAIL]], rows_v.at[TAIL],
                         gsem).wait()
        pltpu.async_copy(rows_v.at[TAIL], accum_sh.at[dst_v.at[TAIL]], ssem,
                         add=True).wait()
    plsc.subcore_barrier()

    def writeback(off, sz):
        pltpu.sync_copy(accum_sh.at[pl.ds(off, sz)],
                        out_hbm.at[c, pl.ds(off, sz)])

    _stage_slices(s, writeback)


NR, NL = 80, 1280
BR = 16
F = 16


def _tc_dense1_body(p_ref, x_ref, wl1_ref, bl1_ref, wr1_ref, wl2_ref,
                    bl2_ref, wr2_ref, z_ref, r_ref, deg_ref):
    p = p_ref[...]
    ssum = p[0] + p[1]
    deg = jnp.maximum(ssum[2], 1.0)
    m0 = ssum[0] / deg
    m1 = ssum[1] / deg
    x0 = x_ref[0]
    x1 = x_ref[1]
    zacc = jnp.zeros_like(m0)
    racc = jnp.zeros_like(m0)
    for f in range(F):
        hf = (m0 * wl1_ref[f, 0] + m1 * wl1_ref[f, 1] + bl1_ref[f]
              + x0 * wr1_ref[f, 0] + x1 * wr1_ref[f, 1])
        hf = jnp.maximum(hf, 0.0)
        zacc = zacc + hf * wl2_ref[0, f]
        racc = racc + hf * wr2_ref[0, f]
    z_ref[...] = zacc
    r_ref[...] = racc + bl2_ref[0]
    deg_ref[...] = deg


def _tc_dense2_body(pz_ref, deg_ref, r_ref, out_ref):
    pz = pz_ref[...]
    out_ref[...] = jax.nn.sigmoid((pz[0] + pz[1]) / deg_ref[...] + r_ref[...])


def kernel(x, edge_index, Wl1, bl1, Wr1, Wl2, bl2, Wr2):
    f32 = jnp.float32
    x = x.astype(f32)
    e3 = (edge_index.astype(jnp.int32)
          .reshape(2, NG, G).transpose(1, 0, 2))
    x8 = jnp.concatenate(
        [x, jnp.ones((N_NODES, 1), f32), jnp.zeros((N_NODES, 5), f32)], axis=1)
    x8 = jnp.pad(x8, ((0, N_PAD - N_NODES), (0, 0)))
    zeros8 = jnp.zeros((N_PAD, 8), f32)

    part1 = _sc_pass1(x8, e3, zeros8)
    p_t = part1.transpose(0, 2, 1).reshape(NC, 8, NR, NL)
    x_t = jnp.pad(x.T, ((0, 0), (0, N_PAD - N_NODES))).reshape(2, NR, NL)

    smem = pltpu.SMEM
    grid = (NR // BR,)
    z, r, deg = pl.pallas_call(
        _tc_dense1_body,
        grid=grid,
        in_specs=[
            pl.BlockSpec((NC, 3, BR, NL), lambda i: (0, 0, i, 0)),
            pl.BlockSpec((2, BR, NL), lambda i: (0, i, 0)),
            pl.BlockSpec(memory_space=smem),
            pl.BlockSpec(memory_space=smem),
            pl.BlockSpec(memory_space=smem),
            pl.BlockSpec(memory_space=smem),
            pl.BlockSpec(memory_space=smem),
            pl.BlockSpec(memory_space=smem),
        ],
        out_specs=[
            pl.BlockSpec((BR, NL), lambda i: (i, 0)),
            pl.BlockSpec((BR, NL), lambda i: (i, 0)),
            pl.BlockSpec((BR, NL), lambda i: (i, 0)),
        ],
        out_shape=[
            jax.ShapeDtypeStruct((NR, NL), f32),
            jax.ShapeDtypeStruct((NR, NL), f32),
            jax.ShapeDtypeStruct((NR, NL), f32),
        ],
    )(p_t, x_t, Wl1, bl1, Wr1, Wl2, bl2, Wr2)

    z8 = jnp.concatenate(
        [z.reshape(N_PAD, 1), jnp.zeros((N_PAD, 7), f32)], axis=1)
    part2 = _sc_pass2(z8, e3, zeros8)
    pz = part2[:, :, 0].reshape(NC, NR, NL)

    out = pl.pallas_call(
        _tc_dense2_body,
        grid=grid,
        in_specs=[
            pl.BlockSpec((NC, BR, NL), lambda i: (0, i, 0)),
            pl.BlockSpec((BR, NL), lambda i: (i, 0)),
            pl.BlockSpec((BR, NL), lambda i: (i, 0)),
        ],
        out_specs=pl.BlockSpec((BR, NL), lambda i: (i, 0)),
        out_shape=jax.ShapeDtypeStruct((NR, NL), f32),
    )(pz, deg, r)
    return out.reshape(N_PAD)[:N_NODES]


# speedup vs baseline: 1.6938x; 1.0528x over previous
import functools

import jax
import jax.numpy as jnp
from jax import lax
from jax.experimental import pallas as pl
from jax.experimental.pallas import tpu as pltpu
from jax.experimental.pallas import tpu_sc as plsc

N_NODES = 100000
N_PAD = 102400
N_EDGES = 3200000

NC = 2
NS = 16
NW = NC * NS
G = 128
GP = 8
NG = N_EDGES // G
GROUPS_PER_W = NG // NW
CHUNKS = GROUPS_PER_W // GP
TAIL = GROUPS_PER_W - CHUNKS * GP
NG_EVEN = GROUPS_PER_W * NW

STG = N_PAD // NS

_mesh = plsc.VectorSubcoreMesh(core_axis_name="c", subcore_axis_name="s")
_sc_params = pltpu.CompilerParams(use_tc_tiling_on_sc=False)


def _stage_slices(sid, copy_fn):
    copy_fn(sid * STG, STG)


@functools.partial(
    pl.kernel,
    out_type=jax.ShapeDtypeStruct((NC, N_PAD, 8), jnp.float32),
    mesh=_mesh,
    scratch_types=[
        pltpu.VMEM_SHARED((N_PAD, 8), jnp.float32),
        pltpu.VMEM_SHARED((N_PAD, 8), jnp.float32),
        pltpu.VMEM((2, GP, G), jnp.int32),
        pltpu.VMEM((2, GP, G), jnp.int32),
        pltpu.VMEM((2, GP, G, 8), jnp.float32),
        pltpu.SemaphoreType.DMA,
        pltpu.SemaphoreType.DMA,
    ],
    compiler_params=_sc_params,
)
def _sc_pass1(x4_hbm, edges_hbm, zeros_hbm, out_hbm,
              table_sh, accum_sh, src_v, dst_v, rows_v, gsem, ssem):
    c = lax.axis_index("c")
    s = lax.axis_index("s")
    w = s * NC + c

    def stage(off, sz):
        pltpu.sync_copy(x4_hbm.at[pl.ds(off, sz)], table_sh.at[pl.ds(off, sz)])
        pltpu.sync_copy(zeros_hbm.at[pl.ds(off, sz)], accum_sh.at[pl.ds(off, sz)])

    _stage_slices(s, stage)
    plsc.subcore_barrier()

    base = w * GROUPS_PER_W

    def slab(b, goff):
        pltpu.sync_copy(edges_hbm.at[pl.ds(goff, GP), 0], src_v.at[b])
        pltpu.sync_copy(edges_hbm.at[pl.ds(goff, GP), 1], dst_v.at[b])

    def fire_gathers(b):
        for j in range(GP):
            pltpu.async_copy(table_sh.at[src_v.at[b, j]], rows_v.at[b, j],
                             gsem)

    def wait_gathers(b):
        for j in range(GP):
            pltpu.make_async_copy(table_sh.at[src_v.at[b, j]],
                                  rows_v.at[b, j], gsem).wait()

    def fire_scatters(b):
        for j in range(GP):
            pltpu.async_copy(rows_v.at[b, j], accum_sh.at[dst_v.at[b, j]],
                             ssem, add=True)

    def wait_scatters(b):
        for j in range(GP):
            pltpu.make_async_copy(rows_v.at[b, j],
                                  accum_sh.at[dst_v.at[b, j]], ssem).wait()

    slab(0, base)
    fire_gathers(0)

    def chunk(k, carry):
        b = lax.rem(k, 2)
        nb = 1 - b

        @pl.when(k > 0)
        def _():
            wait_scatters(nb)

        @pl.when(k + 1 < CHUNKS)
        def _():
            slab(nb, base + (k + 1) * GP)
            fire_gathers(nb)

        wait_gathers(b)
        fire_scatters(b)
        return carry

    lax.fori_loop(0, CHUNKS, chunk, 0)
    wait_scatters((CHUNKS - 1) % 2)

    goff_t = base + CHUNKS * GP
    pltpu.sync_copy(edges_hbm.at[pl.ds(goff_t, TAIL), 0],
                    src_v.at[0, pl.ds(0, TAIL)])
    pltpu.sync_copy(edges_hbm.at[pl.ds(goff_t, TAIL), 1],
                    dst_v.at[0, pl.ds(0, TAIL)])

    @pl.when(w < NG - NG_EVEN)
    def _():
        pltpu.sync_copy(edges_hbm.at[NG_EVEN + w, 0], src_v.at[0, TAIL])
        pltpu.sync_copy(edges_hbm.at[NG_EVEN + w, 1], dst_v.at[0, TAIL])

    tail_g = [
        pltpu.async_copy(table_sh.at[src_v.at[0, j]], rows_v.at[0, j], gsem)
        for j in range(TAIL)
    ]
    for cp in tail_g:
        cp.wait()
    tail_s = [
        pltpu.async_copy(rows_v.at[0, j], accum_sh.at[dst_v.at[0, j]], ssem,
                         add=True)
        for j in range(TAIL)
    ]
    for cp in tail_s:
        cp.wait()

    @pl.when(w < NG - NG_EVEN)
    def _():
        pltpu.async_copy(table_sh.at[src_v.at[0, TAIL]], rows_v.at[0, TAIL],
                         gsem).wait()
        pltpu.async_copy(rows_v.at[0, TAIL], accum_sh.at[dst_v.at[0, TAIL]],
                         ssem, add=True).wait()
    plsc.subcore_barrier()

    def writeback(off, sz):
        pltpu.sync_copy(accum_sh.at[pl.ds(off, sz)],
                        out_hbm.at[c, pl.ds(off, sz)])

    _stage_slices(s, writeback)


@functools.partial(
    pl.kernel,
    out_type=jax.ShapeDtypeStruct((NC, N_PAD, 8), jnp.float32),
    mesh=_mesh,
    scratch_types=[
        pltpu.VMEM_SHARED((N_PAD, 8), jnp.float32),
        pltpu.VMEM_SHARED((N_PAD, 8), jnp.float32),
        pltpu.VMEM((2, GP, G), jnp.int32),
        pltpu.VMEM((2, GP, G), jnp.int32),
        pltpu.VMEM((2, GP, G, 8), jnp.float32),
        pltpu.SemaphoreType.DMA,
        pltpu.SemaphoreType.DMA,
    ],
    compiler_params=_sc_params,
)
def _sc_pass2(z_hbm, edges_hbm, zeros_hbm, out_hbm,
              table_sh, accum_sh, src_v, dst_v, rows_v, gsem, ssem):
    c = lax.axis_index("c")
    s = lax.axis_index("s")
    w = s * NC + c

    def stage(off, sz):
        pltpu.sync_copy(z_hbm.at[pl.ds(off, sz)], table_sh.at[pl.ds(off, sz)])
        pltpu.sync_copy(zeros_hbm.at[pl.ds(off, sz)], accum_sh.at[pl.ds(off, sz)])

    _stage_slices(s, stage)
    plsc.subcore_barrier()

    base = w * GROUPS_PER_W

    def slab(b, goff):
        pltpu.sync_copy(edges_hbm.at[pl.ds(goff, GP), 0], src_v.at[b])
        pltpu.sync_copy(edges_hbm.at[pl.ds(goff, GP), 1], dst_v.at[b])

    def fire_gathers(b):
        for j in range(GP):
            pltpu.async_copy(table_sh.at[src_v.at[b, j]], rows_v.at[b, j],
                             gsem)

    def wait_gathers(b):
        for j in range(GP):
            pltpu.make_async_copy(table_sh.at[src_v.at[b, j]],
                                  rows_v.at[b, j], gsem).wait()

    def fire_scatters(b):
        for j in range(GP):
            pltpu.async_copy(rows_v.at[b, j], accum_sh.at[dst_v.at[b, j]],
                             ssem, add=True)

    def wait_scatters(b):
        for j in range(GP):
            pltpu.make_async_copy(rows_v.at[b, j],
                                  accum_sh.at[dst_v.at[b, j]], ssem).wait()

    slab(0, base)
    fire_gathers(0)

    def chunk(k, carry):
        b = lax.rem(k, 2)
        nb = 1 - b

        @pl.when(k > 0)
        def _():
            wait_scatters(nb)

        @pl.when(k + 1 < CHUNKS)
        def _():
            slab(nb, base + (k + 1) * GP)
            fire_gathers(nb)

        wait_gathers(b)
        fire_scatters(b)
        return carry

    lax.fori_loop(0, CHUNKS, chunk, 0)
    wait_scatters((CHUNKS - 1) % 2)

    goff_t = base + CHUNKS * GP
    pltpu.sync_copy(edges_hbm.at[pl.ds(goff_t, TAIL), 0],
                    src_v.at[0, pl.ds(0, TAIL)])
    pltpu.sync_copy(edges_hbm.at[pl.ds(goff_t, TAIL), 1],
                    dst_v.at[0, pl.ds(0, TAIL)])

    @pl.when(w < NG - NG_EVEN)
    def _():
        pltpu.sync_copy(edges_hbm.at[NG_EVEN + w, 0], src_v.at[0, TAIL])
        pltpu.sync_copy(edges_hbm.at[NG_EVEN + w, 1], dst_v.at[0, TAIL])

    tail_g = [
        pltpu.async_copy(table_sh.at[src_v.at[0, j]], rows_v.at[0, j], gsem)
        for j in range(TAIL)
    ]
    for cp in tail_g:
        cp.wait()
    tail_s = [
        pltpu.async_copy(rows_v.at[0, j], accum_sh.at[dst_v.at[0, j]], ssem,
                         add=True)
        for j in range(TAIL)
    ]
    for cp in tail_s:
        cp.wait()

    @pl.when(w < NG - NG_EVEN)
    def _():
        pltpu.async_copy(table_sh.at[src_v.at[0, TAIL]], rows_v.at[0, TAIL],
                         gsem).wait()
        pltpu.async_copy(rows_v.at[0, TAIL], accum_sh.at[dst_v.at[0, TAIL]],
                         ssem, add=True).wait()
    plsc.subcore_barrier()

    def writeback(off, sz):
        pltpu.sync_copy(accum_sh.at[pl.ds(off, sz)],
                        out_hbm.at[c, pl.ds(off, sz)])

    _stage_slices(s, writeback)


NR, NL = 80, 1280
BR = 16
F = 16


def _tc_dense1_body(p_ref, x_ref, wl1_ref, bl1_ref, wr1_ref, wl2_ref,
                    bl2_ref, wr2_ref, z_ref, r_ref, deg_ref):
    p = p_ref[...]
    ssum = p[0] + p[1]
    deg = jnp.maximum(ssum[2], 1.0)
    m0 = ssum[0] / deg
    m1 = ssum[1] / deg
    x0 = x_ref[0]
    x1 = x_ref[1]
    zacc = jnp.zeros_like(m0)
    racc = jnp.zeros_like(m0)
    for f in range(F):
        hf = (m0 * wl1_ref[f, 0] + m1 * wl1_ref[f, 1] + bl1_ref[f]
              + x0 * wr1_ref[f, 0] + x1 * wr1_ref[f, 1])
        hf = jnp.maximum(hf, 0.0)
        zacc = zacc + hf * wl2_ref[0, f]
        racc = racc + hf * wr2_ref[0, f]
    z_ref[...] = zacc
    r_ref[...] = racc + bl2_ref[0]
    deg_ref[...] = deg


def _tc_dense2_body(pz_ref, deg_ref, r_ref, out_ref):
    pz = pz_ref[...]
    out_ref[...] = jax.nn.sigmoid((pz[0] + pz[1]) / deg_ref[...] + r_ref[...])


def kernel(x, edge_index, Wl1, bl1, Wr1, Wl2, bl2, Wr2):
    f32 = jnp.float32
    x = x.astype(f32)
    e3 = (edge_index.astype(jnp.int32)
          .reshape(2, NG, G).transpose(1, 0, 2))
    x8 = jnp.concatenate(
        [x, jnp.ones((N_NODES, 1), f32), jnp.zeros((N_NODES, 5), f32)], axis=1)
    x8 = jnp.pad(x8, ((0, N_PAD - N_NODES), (0, 0)))
    zeros8 = jnp.zeros((N_PAD, 8), f32)

    part1 = _sc_pass1(x8, e3, zeros8)
    p_t = part1.transpose(0, 2, 1).reshape(NC, 8, NR, NL)
    x_t = jnp.pad(x.T, ((0, 0), (0, N_PAD - N_NODES))).reshape(2, NR, NL)

    smem = pltpu.SMEM
    grid = (NR // BR,)
    z, r, deg = pl.pallas_call(
        _tc_dense1_body,
        grid=grid,
        in_specs=[
            pl.BlockSpec((NC, 3, BR, NL), lambda i: (0, 0, i, 0)),
            pl.BlockSpec((2, BR, NL), lambda i: (0, i, 0)),
            pl.BlockSpec(memory_space=smem),
            pl.BlockSpec(memory_space=smem),
            pl.BlockSpec(memory_space=smem),
            pl.BlockSpec(memory_space=smem),
            pl.BlockSpec(memory_space=smem),
            pl.BlockSpec(memory_space=smem),
        ],
        out_specs=[
            pl.BlockSpec((BR, NL), lambda i: (i, 0)),
            pl.BlockSpec((BR, NL), lambda i: (i, 0)),
            pl.BlockSpec((BR, NL), lambda i: (i, 0)),
        ],
        out_shape=[
            jax.ShapeDtypeStruct((NR, NL), f32),
            jax.ShapeDtypeStruct((NR, NL), f32),
            jax.ShapeDtypeStruct((NR, NL), f32),
        ],
    )(p_t, x_t, Wl1, bl1, Wr1, Wl2, bl2, Wr2)

    z8 = jnp.concatenate(
        [z.reshape(N_PAD, 1), jnp.zeros((N_PAD, 7), f32)], axis=1)
    part2 = _sc_pass2(z8, e3, zeros8)
    pz = part2[:, :, 0].reshape(NC, NR, NL)

    out = pl.pallas_call(
        _tc_dense2_body,
        grid=grid,
        in_specs=[
            pl.BlockSpec((NC, BR, NL), lambda i: (0, i, 0)),
            pl.BlockSpec((BR, NL), lambda i: (i, 0)),
            pl.BlockSpec((BR, NL), lambda i: (i, 0)),
        ],
        out_specs=pl.BlockSpec((BR, NL), lambda i: (i, 0)),
        out_shape=jax.ShapeDtypeStruct((NR, NL), f32),
    )(pz, deg, r)
    return out.reshape(N_PAD)[:N_NODES]
